# Initial kernel scaffold; baseline (speedup 1.0000x reference)
import jax
import jax.numpy as jnp
from jax.experimental import pallas as pl


def kernel(X, X_target, theta, bias, H_row, H_col):
    raise NotImplementedError("write your pallas kernel here")


# speedup vs baseline: 8.1559x; 8.1559x over previous
import functools

import jax
import jax.numpy as jnp
from jax import lax
from jax.experimental import pallas as pl
from jax.experimental.pallas import tpu as pltpu
from jax.experimental.pallas import tpu_sc as plsc

N = 10000
E = 5000
NNZ = 320000
D = 128

NC = 2
NS = 16
L = 16
NW = NC * NS
DH = D // 2
W = DH + L
NV = W // L

N_PAD = 10240
E_PAD = 5120
BATCH = 128
NB = 160
NNZ_PAD = NS * NB * BATCH

_mesh = plsc.VectorSubcoreMesh(
    core_axis_name="c", subcore_axis_name="s", num_cores=NC, num_subcores=NS
)
_sc_params = pltpu.CompilerParams(use_tc_tiling_on_sc=False)


def _worker_id():
    return lax.axis_index("s") * NC + lax.axis_index("c")


def _mm_body(x_ref, t_ref, o_ref):
    o_ref[...] = jnp.dot(x_ref[...], t_ref[...],
                         preferred_element_type=jnp.float32)


def _matmul(x_pad, theta):
    bm = 256
    return pl.pallas_call(
        _mm_body,
        grid=(N_PAD // bm,),
        in_specs=[
            pl.BlockSpec((bm, D), lambda i: (i, 0)),
            pl.BlockSpec((D, D), lambda i: (0, 0)),
        ],
        out_specs=pl.BlockSpec((bm, D), lambda i: (i, 0)),
        out_shape=jax.ShapeDtypeStruct((N_PAD, D), jnp.float32),
    )(x_pad, theta)


def _make_pass(tab_rows, acc_rows):
    rows_per_sub = acc_rows // NS
    zr = 160

    @functools.partial(
        pl.kernel,
        out_type=jax.ShapeDtypeStruct((NC, acc_rows, W), jnp.float32),
        mesh=_mesh,
        scratch_types=[
            pltpu.VMEM((NB, BATCH), jnp.int32),
            pltpu.VMEM((NB, BATCH), jnp.int32),
            pltpu.VMEM((2, BATCH, W), jnp.float32),
            pltpu.VMEM((zr, W), jnp.float32),
            pltpu.VMEM_SHARED((acc_rows, W), jnp.float32),
            pltpu.SemaphoreType.DMA,
            pltpu.SemaphoreType.DMA,
        ],
        compiler_params=_sc_params,
    )
    def k(tab_hbm, gidx_hbm, sidx_hbm, out_hbm,
          gidx_v, sidx_v, buf, zbuf, acc, sem0, sem1):
        cid = lax.axis_index("c")
        sid = lax.axis_index("s")
        sems = (sem0, sem1)

        def zrow(r, carry):
            for j in range(NV):
                zbuf[r, pl.ds(j * L, L)] = jnp.zeros((L,), jnp.float32)
            return carry
        lax.fori_loop(0, zr, zrow, 0)
        for cpy in range(rows_per_sub // zr):
            pltpu.sync_copy(
                zbuf, acc.at[pl.ds(sid * rows_per_sub + cpy * zr, zr)])

        pltpu.sync_copy(gidx_hbm.at[sid], gidx_v)
        pltpu.sync_copy(sidx_hbm.at[sid], sidx_v)
        off = jnp.broadcast_to(cid * tab_rows, (L,)).astype(jnp.int32)

        def add_off(r, carry):
            for j in range(BATCH // L):
                gidx_v[r, pl.ds(j * L, L)] = gidx_v[r, pl.ds(j * L, L)] + off
            return carry
        lax.fori_loop(0, NB, add_off, 0)
        plsc.subcore_barrier()

        def start(j, slot):
            pltpu.async_copy(tab_hbm.at[gidx_v.at[j]], buf.at[slot],
                             sems[slot])

        def wait(slot):
            pltpu.make_async_copy(tab_hbm.at[gidx_v.at[0]], buf.at[slot],
                                  sems[slot]).wait()

        start(0, 0)

        def body(i, carry):
            j0 = 2 * i
            start(j0 + 1, 1)
            wait(0)
            pltpu.sync_copy(buf.at[0], acc.at[sidx_v.at[j0]], add=True)

            @pl.when(i < NB // 2 - 1)
            def _():
                start(j0 + 2, 0)

            wait(1)
            pltpu.sync_copy(buf.at[1], acc.at[sidx_v.at[j0 + 1]], add=True)
            return carry
        lax.fori_loop(0, NB // 2, body, 0)

        plsc.subcore_barrier()
        pltpu.sync_copy(
            acc.at[pl.ds(sid * rows_per_sub, rows_per_sub)],
            out_hbm.at[cid, pl.ds(sid * rows_per_sub, rows_per_sub)])

    return k


_pass_a = _make_pass(N_PAD, E_PAD)
_pass_b = _make_pass(E_PAD, N_PAD)


@functools.partial(
    pl.kernel,
    out_type=jax.ShapeDtypeStruct((NC * E_PAD, W), jnp.float32),
    mesh=_mesh,
    scratch_types=[pltpu.VMEM((160, W), jnp.float32)],
    compiler_params=_sc_params,
)
def _scale(part_hbm, out_hbm, buf):
    wid = _worker_id()
    for half in range(2):
        base = wid * 320 + half * 160
        pltpu.sync_copy(part_hbm.at[pl.ds(base, 160)], buf)

        def row(r, carry):
            s = [buf[r, pl.ds(j * L, L)] for j in range(NV)]
            cnt = s[NV - 1]
            pos = cnt > 0.0
            norm = 1.0 / jnp.where(pos, cnt, 1.0)
            for j in range(NV - 1):
                buf[r, pl.ds(j * L, L)] = s[j] * norm
            buf[r, pl.ds(DH, L)] = jnp.where(pos, 1.0, 0.0)
            return carry
        lax.fori_loop(0, 160, row, 0)
        pltpu.sync_copy(buf, out_hbm.at[pl.ds(base, 160)])


@functools.partial(
    pl.kernel,
    out_type=jax.ShapeDtypeStruct((N_PAD, D), jnp.float32),
    mesh=_mesh,
    scratch_types=[
        pltpu.VMEM((160, W), jnp.float32),
        pltpu.VMEM((160, W), jnp.float32),
        pltpu.VMEM((160, D), jnp.float32),
        pltpu.VMEM((D,), jnp.float32),
    ],
    compiler_params=_sc_params,
)
def _final(part_hbm, bias_hbm, out_hbm, buf_l, buf_h, buf_o, bias_v):
    wid = _worker_id()
    pltpu.sync_copy(bias_hbm, bias_v)
    for half in range(2):
        base = wid * 320 + half * 160
        pltpu.sync_copy(part_hbm.at[0, pl.ds(base, 160)], buf_l)
        pltpu.sync_copy(part_hbm.at[1, pl.ds(base, 160)], buf_h)

        def row(r, carry):
            cnt = buf_l[r, pl.ds(DH, L)]
            pos = cnt > 0.0
            norm = 1.0 / jnp.where(pos, cnt, 1.0)
            norm = jnp.where(pos, norm, 0.0)
            for j in range(NV - 1):
                buf_o[r, pl.ds(j * L, L)] = (
                    buf_l[r, pl.ds(j * L, L)] * norm
                    + bias_v[pl.ds(j * L, L)])
                buf_o[r, pl.ds(DH + j * L, L)] = (
                    buf_h[r, pl.ds(j * L, L)] * norm
                    + bias_v[pl.ds(DH + j * L, L)])
            return carry
        lax.fori_loop(0, 160, row, 0)
        pltpu.sync_copy(buf_o, out_hbm.at[pl.ds(base, 160)])


def kernel(X, X_target, theta, bias, H_row, H_col):
    del X_target
    x_pad = jnp.zeros((N_PAD, D), jnp.float32).at[:N].set(X)
    xp = _matmul(x_pad, theta)
    ones = jnp.ones((N_PAD, L), jnp.float32)
    xp2 = jnp.concatenate([
        jnp.concatenate([xp[:, :DH], ones], axis=1),
        jnp.concatenate([xp[:, DH:], ones], axis=1),
    ], axis=0)

    pad_n = NNZ_PAD - NNZ
    hr = jnp.concatenate(
        [H_row, jnp.full((pad_n,), N_PAD - 1, jnp.int32)]
    ).reshape(NS, NB, BATCH)
    hc = jnp.concatenate(
        [H_col, jnp.full((pad_n,), E_PAD - 1, jnp.int32)]
    ).reshape(NS, NB, BATCH)

    y_part = _pass_a(xp2, hr, hc)
    y_aug = _scale(y_part.reshape(NC * E_PAD, W))
    x_part = _pass_b(y_aug, hc, hr)
    out = _final(x_part, bias)
    return out[:N]
